# Initial kernel scaffold; baseline (speedup 1.0000x reference)
import jax
import jax.numpy as jnp
from jax.experimental import pallas as pl


def kernel(atom_idx, aa_idx, hyb_idx, cont_feat, atom_table, aa_table, hyb_table):
    raise NotImplementedError("write your pallas kernel here")


# speedup vs baseline: 1.0117x; 1.0117x over previous
import functools

import jax
import jax.numpy as jnp
from jax import lax
from jax.experimental import pallas as pl
from jax.experimental.pallas import tpu as pltpu
from jax.experimental.pallas import tpu_sc as plsc

N = 100000
CONT_DIM = 32
EMB_ATOM = 16
EMB_AA = 8
EMB_HYB = 8
OUT_DIM = EMB_ATOM + EMB_AA + EMB_HYB + CONT_DIM

NUM_WORKERS = 32
CHUNK = 1000
NUM_CHUNKS = N // CHUNK
CHUNKS_PER_WORKER = -(-NUM_CHUNKS // NUM_WORKERS)

_mesh = plsc.VectorSubcoreMesh(core_axis_name="c", subcore_axis_name="s")


@functools.partial(
    pl.kernel,
    out_type=jax.ShapeDtypeStruct((N, OUT_DIM), jnp.float32),
    mesh=_mesh,
    scratch_types=[
        pltpu.VMEM((CHUNK,), jnp.int32),
        pltpu.VMEM((CHUNK,), jnp.int32),
        pltpu.VMEM((CHUNK,), jnp.int32),
        pltpu.VMEM((CHUNK, EMB_ATOM), jnp.float32),
        pltpu.VMEM((CHUNK, EMB_AA), jnp.float32),
        pltpu.VMEM((CHUNK, EMB_HYB), jnp.float32),
        pltpu.VMEM((CHUNK, CONT_DIM), jnp.float32),
        pltpu.SemaphoreType.DMA,
    ],
    compiler_params=pltpu.CompilerParams(use_tc_tiling_on_sc=False),
)
def _atom_embed_sc(atom_idx, aa_idx, hyb_idx, cont, atom_tab, aa_tab,
                   hyb_tab, out, ia_v, ib_v, ih_v, a_v, b_v, h_v, c_v, sem):
    wid = lax.axis_index("s") * 2 + lax.axis_index("c")

    def do_chunk(t, _):
        chunk = wid + NUM_WORKERS * t

        @pl.when(chunk < NUM_CHUNKS)
        def _():
            base = chunk * CHUNK
            rows = pl.ds(base, CHUNK)
            pltpu.sync_copy(atom_idx.at[rows], ia_v)
            pltpu.sync_copy(aa_idx.at[rows], ib_v)
            pltpu.sync_copy(hyb_idx.at[rows], ih_v)
            pltpu.async_copy(atom_tab.at[ia_v], a_v, sem).wait()
            pltpu.async_copy(aa_tab.at[ib_v], b_v, sem).wait()
            pltpu.async_copy(hyb_tab.at[ih_v], h_v, sem).wait()
            pltpu.sync_copy(cont.at[rows], c_v)
            pltpu.sync_copy(a_v, out.at[rows, pl.ds(0, EMB_ATOM)])
            pltpu.sync_copy(b_v, out.at[rows, pl.ds(EMB_ATOM, EMB_AA)])
            pltpu.sync_copy(h_v, out.at[rows, pl.ds(EMB_ATOM + EMB_AA, EMB_HYB)])
            pltpu.sync_copy(c_v, out.at[rows, pl.ds(EMB_ATOM + EMB_AA + EMB_HYB, CONT_DIM)])
        return 0

    lax.fori_loop(0, CHUNKS_PER_WORKER, do_chunk, 0)


def kernel(atom_idx, aa_idx, hyb_idx, cont_feat, atom_table, aa_table, hyb_table):
    return _atom_embed_sc(
        atom_idx.astype(jnp.int32),
        aa_idx.astype(jnp.int32),
        hyb_idx.astype(jnp.int32),
        cont_feat,
        atom_table,
        aa_table,
        hyb_table,
    )


# speedup vs baseline: 1.5342x; 1.5164x over previous
import functools

import jax
import jax.numpy as jnp
from jax import lax
from jax.experimental import pallas as pl
from jax.experimental.pallas import tpu as pltpu
from jax.experimental.pallas import tpu_sc as plsc

N = 100000
CONT_DIM = 32
N_ATOMS = 44
N_AA = 22
N_HYB = 8
EMB_ATOM = 16
EMB_AA = 8
EMB_HYB = 8
EMB_DIM = EMB_ATOM + EMB_AA + EMB_HYB
OUT_DIM = EMB_DIM + CONT_DIM
FUSED_ROWS = N_ATOMS * N_AA * N_HYB

NUM_WORKERS = 32
CHUNK = 800
NUM_CHUNKS = N // CHUNK
CHUNKS_PER_WORKER = -(-NUM_CHUNKS // NUM_WORKERS)
LANES = 16

_mesh = plsc.VectorSubcoreMesh(core_axis_name="c", subcore_axis_name="s")


@functools.partial(
    pl.kernel,
    out_type=jax.ShapeDtypeStruct((N, OUT_DIM), jnp.float32),
    mesh=_mesh,
    scratch_types=[
        pltpu.VMEM((CHUNK,), jnp.int32),
        pltpu.VMEM((CHUNK,), jnp.int32),
        pltpu.VMEM((CHUNK,), jnp.int32),
        pltpu.VMEM((CHUNK,), jnp.int32),
        pltpu.VMEM((CHUNK, EMB_DIM), jnp.float32),
        pltpu.SemaphoreType.DMA,
        pltpu.SemaphoreType.DMA,
        pltpu.SemaphoreType.DMA,
        pltpu.SemaphoreType.DMA,
    ],
    compiler_params=pltpu.CompilerParams(use_tc_tiling_on_sc=False),
)
def _atom_embed_sc(atom_idx, aa_idx, hyb_idx, cont, fused_tab, out,
                   ia_v, ib_v, ih_v, fi_v, g_v,
                   sem_i, sem_g, sem_o, sem_c):
    wid = lax.axis_index("s") * 2 + lax.axis_index("c")

    def do_chunk(t, _):
        chunk = wid + NUM_WORKERS * t

        @pl.when(chunk < NUM_CHUNKS)
        def _():
            base = chunk * CHUNK
            rows = pl.ds(base, CHUNK)
            cont_cp = pltpu.async_copy(
                cont.at[rows], out.at[rows, pl.ds(EMB_DIM, CONT_DIM)], sem_c)
            ca = pltpu.async_copy(atom_idx.at[rows], ia_v, sem_i)
            cb = pltpu.async_copy(aa_idx.at[rows], ib_v, sem_i)
            ch = pltpu.async_copy(hyb_idx.at[rows], ih_v, sem_i)
            ca.wait()
            cb.wait()
            ch.wait()

            def fuse(i, _):
                s = pl.ds(i * LANES, LANES)
                fi_v[s] = (ia_v[s] * (N_AA * N_HYB) + ib_v[s] * N_HYB
                           + ih_v[s])
                return 0

            lax.fori_loop(0, CHUNK // LANES, fuse, 0)

            pltpu.async_copy(fused_tab.at[fi_v], g_v, sem_g).wait()
            pltpu.async_copy(g_v, out.at[rows, pl.ds(0, EMB_DIM)], sem_o).wait()
            cont_cp.wait()
        return 0

    lax.fori_loop(0, CHUNKS_PER_WORKER, do_chunk, 0)


def kernel(atom_idx, aa_idx, hyb_idx, cont_feat, atom_table, aa_table, hyb_table):
    fused_tab = jnp.concatenate(
        [
            jnp.broadcast_to(atom_table[:, None, None, :],
                             (N_ATOMS, N_AA, N_HYB, EMB_ATOM)),
            jnp.broadcast_to(aa_table[None, :, None, :],
                             (N_ATOMS, N_AA, N_HYB, EMB_AA)),
            jnp.broadcast_to(hyb_table[None, None, :, :],
                             (N_ATOMS, N_AA, N_HYB, EMB_HYB)),
        ],
        axis=-1,
    ).reshape(FUSED_ROWS, EMB_DIM)
    return _atom_embed_sc(
        atom_idx.astype(jnp.int32),
        aa_idx.astype(jnp.int32),
        hyb_idx.astype(jnp.int32),
        cont_feat,
        fused_tab,
    )


# speedup vs baseline: 4.1295x; 2.6917x over previous
import functools

import jax
import jax.numpy as jnp
from jax import lax
from jax.experimental import pallas as pl
from jax.experimental.pallas import tpu as pltpu
from jax.experimental.pallas import tpu_sc as plsc

N = 100000
CONT_DIM = 32
N_ATOMS = 44
N_AA = 22
N_HYB = 8
EMB_ATOM = 16
EMB_AA = 8
EMB_HYB = 8
EMB_DIM = EMB_ATOM + EMB_AA + EMB_HYB
OUT_DIM = EMB_DIM + CONT_DIM
FUSED_ROWS = N_ATOMS * N_AA * N_HYB

NUM_WORKERS = 32
CHUNK = 800
NUM_CHUNKS = N // CHUNK
CHUNKS_PER_WORKER = -(-NUM_CHUNKS // NUM_WORKERS)
LANES = 16

_mesh = plsc.VectorSubcoreMesh(core_axis_name="c", subcore_axis_name="s")


@functools.partial(
    pl.kernel,
    out_type=jax.ShapeDtypeStruct((N, OUT_DIM), jnp.float32),
    mesh=_mesh,
    scratch_types=[
        pltpu.VMEM((CHUNK,), jnp.int32),
        pltpu.VMEM((CHUNK,), jnp.int32),
        pltpu.VMEM((CHUNK,), jnp.int32),
        pltpu.VMEM((CHUNK,), jnp.int32),
        pltpu.VMEM((CHUNK, OUT_DIM), jnp.float32),
        pltpu.VMEM((CHUNK, EMB_DIM), jnp.float32),
        pltpu.SemaphoreType.DMA,
        pltpu.SemaphoreType.DMA,
        pltpu.SemaphoreType.DMA,
        pltpu.SemaphoreType.DMA,
    ],
    compiler_params=pltpu.CompilerParams(use_tc_tiling_on_sc=False),
)
def _atom_embed_sc(atom_idx, aa_idx, hyb_idx, cont, fused_tab, out,
                   ia_v, ib_v, ih_v, fi_v, ob_v, g_v,
                   sem_i, sem_g, sem_o, sem_c):
    wid = lax.axis_index("s") * 2 + lax.axis_index("c")

    def do_chunk(t, _):
        chunk = wid + NUM_WORKERS * t

        @pl.when(chunk < NUM_CHUNKS)
        def _():
            base = chunk * CHUNK
            rows = pl.ds(base, CHUNK)
            cont_cp = pltpu.async_copy(
                cont.at[rows], ob_v.at[:, pl.ds(EMB_DIM, CONT_DIM)], sem_c)
            ca = pltpu.async_copy(atom_idx.at[rows], ia_v, sem_i)
            cb = pltpu.async_copy(aa_idx.at[rows], ib_v, sem_i)
            ch = pltpu.async_copy(hyb_idx.at[rows], ih_v, sem_i)
            ca.wait()
            cb.wait()
            ch.wait()

            def fuse(i, _):
                s = pl.ds(i * LANES, LANES)
                fi_v[s] = (ia_v[s] * (N_AA * N_HYB) + ib_v[s] * N_HYB
                           + ih_v[s])
                return 0

            lax.fori_loop(0, CHUNK // LANES, fuse, 0)

            pltpu.async_copy(fused_tab.at[fi_v], g_v, sem_g).wait()

            def inter(r, _):
                ob_v[r, pl.ds(0, LANES)] = g_v[r, pl.ds(0, LANES)]
                ob_v[r, pl.ds(LANES, LANES)] = g_v[r, pl.ds(LANES, LANES)]
                return 0

            lax.fori_loop(0, CHUNK, inter, 0)
            cont_cp.wait()
            pltpu.async_copy(ob_v, out.at[rows], sem_o).wait()
        return 0

    lax.fori_loop(0, CHUNKS_PER_WORKER, do_chunk, 0)


def kernel(atom_idx, aa_idx, hyb_idx, cont_feat, atom_table, aa_table, hyb_table):
    fused_tab = jnp.concatenate(
        [
            jnp.broadcast_to(atom_table[:, None, None, :],
                             (N_ATOMS, N_AA, N_HYB, EMB_ATOM)),
            jnp.broadcast_to(aa_table[None, :, None, :],
                             (N_ATOMS, N_AA, N_HYB, EMB_AA)),
            jnp.broadcast_to(hyb_table[None, None, :, :],
                             (N_ATOMS, N_AA, N_HYB, EMB_HYB)),
        ],
        axis=-1,
    ).reshape(FUSED_ROWS, EMB_DIM)
    return _atom_embed_sc(
        atom_idx.astype(jnp.int32),
        aa_idx.astype(jnp.int32),
        hyb_idx.astype(jnp.int32),
        cont_feat,
        fused_tab,
    )


# speedup vs baseline: 4.4470x; 1.0769x over previous
import functools

import jax
import jax.numpy as jnp
from jax import lax
from jax.experimental import pallas as pl
from jax.experimental.pallas import tpu as pltpu
from jax.experimental.pallas import tpu_sc as plsc

N = 100000
CONT_DIM = 32
N_ATOMS = 44
N_AA = 22
N_HYB = 8
EMB_ATOM = 16
EMB_AA = 8
EMB_HYB = 8
EMB_DIM = EMB_ATOM + EMB_AA + EMB_HYB
OUT_DIM = EMB_DIM + CONT_DIM
FUSED_ROWS = N_ATOMS * N_AA * N_HYB

NUM_WORKERS = 32
CHUNK = 800
NUM_CHUNKS = N // CHUNK
CHUNKS_PER_WORKER = -(-NUM_CHUNKS // NUM_WORKERS)
LANES = 16
PACK = 128 // EMB_DIM
PACKED_ROWS = N // PACK
CHUNK_PACKED = CHUNK // PACK

_mesh = plsc.VectorSubcoreMesh(core_axis_name="c", subcore_axis_name="s")


@functools.partial(
    pl.kernel,
    out_type=jax.ShapeDtypeStruct((PACKED_ROWS, 128), jnp.float32),
    mesh=_mesh,
    scratch_types=[
        pltpu.VMEM((CHUNK,), jnp.int32),
        pltpu.VMEM((CHUNK,), jnp.int32),
        pltpu.VMEM((CHUNK,), jnp.int32),
        pltpu.VMEM((CHUNK,), jnp.int32),
        pltpu.VMEM((CHUNK, EMB_DIM), jnp.float32),
        pltpu.VMEM((CHUNK_PACKED, 128), jnp.float32),
        pltpu.SemaphoreType.DMA,
        pltpu.SemaphoreType.DMA,
        pltpu.SemaphoreType.DMA,
    ],
    compiler_params=pltpu.CompilerParams(use_tc_tiling_on_sc=False),
)
def _gather_sc(atom_idx, aa_idx, hyb_idx, fused_tab, out,
               ia_v, ib_v, ih_v, fi_v, g_v, o_v, sem_i, sem_g, sem_o):
    wid = lax.axis_index("s") * 2 + lax.axis_index("c")

    def do_chunk(t, _):
        chunk = wid + NUM_WORKERS * t

        @pl.when(chunk < NUM_CHUNKS)
        def _():
            base = chunk * CHUNK
            rows = pl.ds(base, CHUNK)
            ca = pltpu.async_copy(atom_idx.at[rows], ia_v, sem_i)
            cb = pltpu.async_copy(aa_idx.at[rows], ib_v, sem_i)
            ch = pltpu.async_copy(hyb_idx.at[rows], ih_v, sem_i)
            ca.wait()
            cb.wait()
            ch.wait()

            def fuse(i, _):
                s = pl.ds(i * LANES, LANES)
                fi_v[s] = (ia_v[s] * (N_AA * N_HYB) + ib_v[s] * N_HYB
                           + ih_v[s])
                return 0

            lax.fori_loop(0, CHUNK // LANES, fuse, 0)

            pltpu.async_copy(fused_tab.at[fi_v], g_v, sem_g).wait()

            def pack_loop(p, _):
                for j in range(PACK):
                    r = p * PACK + j
                    o_v[p, pl.ds(j * EMB_DIM, LANES)] = g_v[r, pl.ds(0, LANES)]
                    o_v[p, pl.ds(j * EMB_DIM + LANES, LANES)] = (
                        g_v[r, pl.ds(LANES, LANES)])
                return 0

            lax.fori_loop(0, CHUNK_PACKED, pack_loop, 0)
            pltpu.async_copy(
                o_v,
                out.at[pl.ds(chunk * CHUNK_PACKED, CHUNK_PACKED)],
                sem_o,
            ).wait()
        return 0

    lax.fori_loop(0, CHUNKS_PER_WORKER, do_chunk, 0)


TC_BLOCK = 4000


def _concat_tc(emb_ref, cont_ref, out_ref):
    out_ref[...] = jnp.concatenate([emb_ref[...], cont_ref[...]], axis=1)


_concat_call = pl.pallas_call(
    _concat_tc,
    grid=(N // TC_BLOCK,),
    in_specs=[
        pl.BlockSpec((TC_BLOCK, EMB_DIM), lambda i: (i, 0)),
        pl.BlockSpec((TC_BLOCK, CONT_DIM), lambda i: (i, 0)),
    ],
    out_specs=pl.BlockSpec((TC_BLOCK, OUT_DIM), lambda i: (i, 0)),
    out_shape=jax.ShapeDtypeStruct((N, OUT_DIM), jnp.float32),
)


def kernel(atom_idx, aa_idx, hyb_idx, cont_feat, atom_table, aa_table, hyb_table):
    fused_tab = jnp.concatenate(
        [
            jnp.broadcast_to(atom_table[:, None, None, :],
                             (N_ATOMS, N_AA, N_HYB, EMB_ATOM)),
            jnp.broadcast_to(aa_table[None, :, None, :],
                             (N_ATOMS, N_AA, N_HYB, EMB_AA)),
            jnp.broadcast_to(hyb_table[None, None, :, :],
                             (N_ATOMS, N_AA, N_HYB, EMB_HYB)),
        ],
        axis=-1,
    ).reshape(FUSED_ROWS, EMB_DIM)
    packed = _gather_sc(
        atom_idx.astype(jnp.int32),
        aa_idx.astype(jnp.int32),
        hyb_idx.astype(jnp.int32),
        fused_tab,
    )
    return _concat_call(packed.reshape(N, EMB_DIM), cont_feat)


# speedup vs baseline: 6.9762x; 1.5687x over previous
import functools

import jax
import jax.numpy as jnp
from jax import lax
from jax.experimental import pallas as pl
from jax.experimental.pallas import tpu as pltpu
from jax.experimental.pallas import tpu_sc as plsc

N = 100000
CONT_DIM = 32
N_ATOMS = 44
N_AA = 22
N_HYB = 8
EMB_ATOM = 16
EMB_AA = 8
EMB_HYB = 8
EMB_DIM = EMB_ATOM + EMB_AA + EMB_HYB
OUT_DIM = EMB_DIM + CONT_DIM
FUSED_ROWS = N_ATOMS * N_AA * N_HYB

NUM_WORKERS = 32
CHUNK = 800
NUM_CHUNKS = N // CHUNK
CHUNKS_PER_WORKER = -(-NUM_CHUNKS // NUM_WORKERS)
LANES = 16
PACK = 128 // EMB_DIM
PACKED_ROWS = N // PACK
CHUNK_PACKED = CHUNK // PACK

_mesh = plsc.VectorSubcoreMesh(core_axis_name="c", subcore_axis_name="s")


@functools.partial(
    pl.kernel,
    out_type=jax.ShapeDtypeStruct((PACKED_ROWS, 128), jnp.float32),
    mesh=_mesh,
    scratch_types=[
        pltpu.VMEM((CHUNK,), jnp.int32),
        pltpu.VMEM((CHUNK,), jnp.int32),
        pltpu.VMEM((CHUNK,), jnp.int32),
        pltpu.VMEM((CHUNK,), jnp.int32),
        pltpu.VMEM((CHUNK, EMB_DIM), jnp.float32),
        pltpu.VMEM((CHUNK_PACKED, 128), jnp.float32),
        pltpu.SemaphoreType.DMA,
        pltpu.SemaphoreType.DMA,
        pltpu.SemaphoreType.DMA,
    ],
    compiler_params=pltpu.CompilerParams(use_tc_tiling_on_sc=False),
)
def _gather_sc(atom_idx, aa_idx, hyb_idx, fused_tab, out,
               ia_v, ib_v, ih_v, fi_v, g_v, o_v, sem_i, sem_g, sem_o):
    wid = lax.axis_index("s") * 2 + lax.axis_index("c")

    def do_chunk(t, _):
        chunk = wid + NUM_WORKERS * t

        @pl.when(chunk < NUM_CHUNKS)
        def _():
            base = chunk * CHUNK
            rows = pl.ds(base, CHUNK)
            ca = pltpu.async_copy(atom_idx.at[rows], ia_v, sem_i)
            cb = pltpu.async_copy(aa_idx.at[rows], ib_v, sem_i)
            ch = pltpu.async_copy(hyb_idx.at[rows], ih_v, sem_i)
            ca.wait()
            cb.wait()
            ch.wait()

            def fuse(i, _):
                s = pl.ds(i * LANES, LANES)
                fi_v[s] = (ia_v[s] * (N_AA * N_HYB) + ib_v[s] * N_HYB
                           + ih_v[s])
                return 0

            lax.fori_loop(0, CHUNK // LANES, fuse, 0)

            pltpu.async_copy(fused_tab.at[fi_v], g_v, sem_g).wait()

            def pack_loop(p, _):
                for j in range(PACK):
                    r = p * PACK + j
                    o_v[p, pl.ds(j * EMB_DIM, LANES)] = g_v[r, pl.ds(0, LANES)]
                    o_v[p, pl.ds(j * EMB_DIM + LANES, LANES)] = (
                        g_v[r, pl.ds(LANES, LANES)])
                return 0

            lax.fori_loop(0, CHUNK_PACKED, pack_loop, 0)
            pltpu.async_copy(
                o_v,
                out.at[pl.ds(chunk * CHUNK_PACKED, CHUNK_PACKED)],
                sem_o,
            ).wait()
        return 0

    lax.fori_loop(0, CHUNKS_PER_WORKER, do_chunk, 0)


TC_BLOCK = 4096


def _concat_tc(emb_ref, cont_ref, out_ref):
    out_ref[0:EMB_DIM, :] = jnp.transpose(emb_ref[...])
    out_ref[EMB_DIM:OUT_DIM, :] = cont_ref[...]


_concat_call = pl.pallas_call(
    _concat_tc,
    grid=(-(-N // TC_BLOCK),),
    in_specs=[
        pl.BlockSpec((TC_BLOCK, EMB_DIM), lambda i: (i, 0)),
        pl.BlockSpec((CONT_DIM, TC_BLOCK), lambda i: (0, i)),
    ],
    out_specs=pl.BlockSpec((OUT_DIM, TC_BLOCK), lambda i: (0, i)),
    out_shape=jax.ShapeDtypeStruct((OUT_DIM, N), jnp.float32),
)


def kernel(atom_idx, aa_idx, hyb_idx, cont_feat, atom_table, aa_table, hyb_table):
    fused_tab = jnp.concatenate(
        [
            jnp.broadcast_to(atom_table[:, None, None, :],
                             (N_ATOMS, N_AA, N_HYB, EMB_ATOM)),
            jnp.broadcast_to(aa_table[None, :, None, :],
                             (N_ATOMS, N_AA, N_HYB, EMB_AA)),
            jnp.broadcast_to(hyb_table[None, None, :, :],
                             (N_ATOMS, N_AA, N_HYB, EMB_HYB)),
        ],
        axis=-1,
    ).reshape(FUSED_ROWS, EMB_DIM)
    packed = _gather_sc(
        atom_idx.astype(jnp.int32),
        aa_idx.astype(jnp.int32),
        hyb_idx.astype(jnp.int32),
        fused_tab,
    )
    out_t = _concat_call(packed.reshape(N, EMB_DIM), cont_feat.T)
    return out_t.T


# speedup vs baseline: 7.1893x; 1.0305x over previous
import functools

import jax
import jax.numpy as jnp
from jax import lax
from jax.experimental import pallas as pl
from jax.experimental.pallas import tpu as pltpu
from jax.experimental.pallas import tpu_sc as plsc

N = 100000
CONT_DIM = 32
N_ATOMS = 44
N_AA = 22
N_HYB = 8
EMB_ATOM = 16
EMB_AA = 8
EMB_HYB = 8
EMB_DIM = EMB_ATOM + EMB_AA + EMB_HYB
OUT_DIM = EMB_DIM + CONT_DIM
FUSED_ROWS = N_ATOMS * N_AA * N_HYB

NUM_WORKERS = 32
CHUNK = 800
NUM_CHUNKS = N // CHUNK
CHUNKS_PER_WORKER = -(-NUM_CHUNKS // NUM_WORKERS)
LANES = 16
PACK = 128 // EMB_DIM
PACKED_ROWS = N // PACK
CHUNK_PACKED = CHUNK // PACK

_mesh = plsc.VectorSubcoreMesh(core_axis_name="c", subcore_axis_name="s")


@functools.partial(
    pl.kernel,
    out_type=jax.ShapeDtypeStruct((N // 8, 8, 128), jnp.float32),
    mesh=_mesh,
    scratch_types=[
        pltpu.VMEM((CHUNK,), jnp.int32),
        pltpu.VMEM((CHUNK,), jnp.int32),
        pltpu.VMEM((CHUNK,), jnp.int32),
        pltpu.VMEM((CHUNK, EMB_DIM), jnp.float32),
        pltpu.VMEM((CHUNK // 8, 8, 128), jnp.float32),
        pltpu.SemaphoreType.DMA,
        pltpu.SemaphoreType.DMA,
        pltpu.SemaphoreType.DMA,
    ],
    compiler_params=pltpu.CompilerParams(use_tc_tiling_on_sc=False),
)
def _gather_sc(atom_idx, aa_idx, hyb_idx, fused_tab, out,
               ia_v, ib_v, ih_v, g_v, o_v, sem_i, sem_g, sem_o):
    wid = lax.axis_index("s") * 2 + lax.axis_index("c")

    def do_chunk(t, _):
        chunk = wid + NUM_WORKERS * t

        @pl.when(chunk < NUM_CHUNKS)
        def _():
            base = chunk * CHUNK
            rows = pl.ds(base, CHUNK)
            ca = pltpu.async_copy(atom_idx.at[rows], ia_v, sem_i)
            cb = pltpu.async_copy(aa_idx.at[rows], ib_v, sem_i)
            ch = pltpu.async_copy(hyb_idx.at[rows], ih_v, sem_i)
            ca.wait()
            cb.wait()
            ch.wait()

            def fuse(i, _):
                s = pl.ds(i * LANES, LANES)
                ia_v[s] = (ia_v[s] * (N_AA * N_HYB) + ib_v[s] * N_HYB
                           + ih_v[s])
                return 0

            lax.fori_loop(0, CHUNK // LANES, fuse, 0)

            pltpu.async_copy(fused_tab.at[ia_v], g_v, sem_g).wait()

            def pack_loop(t, _):
                for s in range(8):
                    r = t * 8 + s
                    o_v[t, s, pl.ds(0, LANES)] = g_v[r, pl.ds(0, LANES)]
                    o_v[t, s, pl.ds(LANES, LANES)] = g_v[r, pl.ds(LANES, LANES)]
                return 0

            lax.fori_loop(0, CHUNK // 8, pack_loop, 0)
            pltpu.async_copy(
                o_v,
                out.at[pl.ds(chunk * (CHUNK // 8), CHUNK // 8)],
                sem_o,
            ).wait()
        return 0

    lax.fori_loop(0, CHUNKS_PER_WORKER, do_chunk, 0)


TC_BLOCK = 4096


def _concat_tc(emb_ref, cont_ref, out_ref):
    e = emb_ref[...].reshape(TC_BLOCK, 128)[:, 0:EMB_DIM]
    out_ref[0:EMB_DIM, :] = jnp.transpose(e)
    out_ref[EMB_DIM:OUT_DIM, :] = cont_ref[...]


_concat_call = pl.pallas_call(
    _concat_tc,
    grid=(-(-N // TC_BLOCK),),
    in_specs=[
        pl.BlockSpec((TC_BLOCK // 8, 8, 128), lambda i: (i, 0, 0)),
        pl.BlockSpec((CONT_DIM, TC_BLOCK), lambda i: (0, i)),
    ],
    out_specs=pl.BlockSpec((OUT_DIM, TC_BLOCK), lambda i: (0, i)),
    out_shape=jax.ShapeDtypeStruct((OUT_DIM, N), jnp.float32),
)


def kernel(atom_idx, aa_idx, hyb_idx, cont_feat, atom_table, aa_table, hyb_table):
    fused_tab = jnp.concatenate(
        [
            jnp.broadcast_to(atom_table[:, None, None, :],
                             (N_ATOMS, N_AA, N_HYB, EMB_ATOM)),
            jnp.broadcast_to(aa_table[None, :, None, :],
                             (N_ATOMS, N_AA, N_HYB, EMB_AA)),
            jnp.broadcast_to(hyb_table[None, None, :, :],
                             (N_ATOMS, N_AA, N_HYB, EMB_HYB)),
        ],
        axis=-1,
    ).reshape(FUSED_ROWS, EMB_DIM)
    packed = _gather_sc(
        atom_idx.astype(jnp.int32),
        aa_idx.astype(jnp.int32),
        hyb_idx.astype(jnp.int32),
        fused_tab,
    )
    out_t = _concat_call(packed, cont_feat.T)
    return out_t.T


# speedup vs baseline: 7.7812x; 1.0823x over previous
import functools

import jax
import jax.numpy as jnp
from jax import lax
from jax.experimental import pallas as pl
from jax.experimental.pallas import tpu as pltpu
from jax.experimental.pallas import tpu_sc as plsc

N = 100000
CONT_DIM = 32
N_ATOMS = 44
N_AA = 22
N_HYB = 8
EMB_ATOM = 16
EMB_AA = 8
EMB_HYB = 8
EMB_DIM = EMB_ATOM + EMB_AA + EMB_HYB
OUT_DIM = EMB_DIM + CONT_DIM
FUSED_ROWS = N_ATOMS * N_AA * N_HYB

NUM_WORKERS = 32
CHUNK = 800
NUM_CHUNKS = N // CHUNK
CHUNKS_PER_WORKER = -(-NUM_CHUNKS // NUM_WORKERS)
LANES = 16
PACK = 128 // EMB_DIM
PACKED_ROWS = N // PACK
CHUNK_PACKED = CHUNK // PACK

_mesh = plsc.VectorSubcoreMesh(core_axis_name="c", subcore_axis_name="s")


@functools.partial(
    pl.kernel,
    out_type=jax.ShapeDtypeStruct((N, 128), jnp.float32),
    mesh=_mesh,
    scratch_types=[
        pltpu.VMEM((CHUNK,), jnp.int32),
        pltpu.VMEM((CHUNK,), jnp.int32),
        pltpu.VMEM((CHUNK,), jnp.int32),
        pltpu.VMEM((CHUNK, 128), jnp.float32),
        pltpu.SemaphoreType.DMA,
        pltpu.SemaphoreType.DMA,
        pltpu.SemaphoreType.DMA,
    ],
    compiler_params=pltpu.CompilerParams(use_tc_tiling_on_sc=False),
)
def _gather_sc(atom_idx, aa_idx, hyb_idx, fused_tab, out,
               ia_v, ib_v, ih_v, o_v, sem_i, sem_g, sem_o):
    wid = lax.axis_index("s") * 2 + lax.axis_index("c")

    def do_chunk(t, _):
        chunk = wid + NUM_WORKERS * t

        @pl.when(chunk < NUM_CHUNKS)
        def _():
            base = chunk * CHUNK
            rows = pl.ds(base, CHUNK)
            ca = pltpu.async_copy(atom_idx.at[rows], ia_v, sem_i)
            cb = pltpu.async_copy(aa_idx.at[rows], ib_v, sem_i)
            ch = pltpu.async_copy(hyb_idx.at[rows], ih_v, sem_i)
            ca.wait()
            cb.wait()
            ch.wait()

            def fuse(i, _):
                s = pl.ds(i * LANES, LANES)
                ia_v[s] = (ia_v[s] * (N_AA * N_HYB) + ib_v[s] * N_HYB
                           + ih_v[s])
                return 0

            lax.fori_loop(0, CHUNK // LANES, fuse, 0)

            pltpu.async_copy(fused_tab.at[ia_v], o_v, sem_g).wait()
            pltpu.async_copy(o_v, out.at[rows], sem_o).wait()
        return 0

    lax.fori_loop(0, CHUNKS_PER_WORKER, do_chunk, 0)


TC_BLOCK = 4096


def _concat_tc(emb_ref, cont_ref, out_ref):
    e = emb_ref[...].reshape(TC_BLOCK, 128)[:, 0:EMB_DIM]
    out_ref[0:EMB_DIM, :] = jnp.transpose(e)
    out_ref[EMB_DIM:OUT_DIM, :] = cont_ref[...]


_concat_call = pl.pallas_call(
    _concat_tc,
    grid=(-(-N // TC_BLOCK),),
    in_specs=[
        pl.BlockSpec((TC_BLOCK // 8, 8, 128), lambda i: (i, 0, 0)),
        pl.BlockSpec((CONT_DIM, TC_BLOCK), lambda i: (0, i)),
    ],
    out_specs=pl.BlockSpec((OUT_DIM, TC_BLOCK), lambda i: (0, i)),
    out_shape=jax.ShapeDtypeStruct((OUT_DIM, N), jnp.float32),
)


def kernel(atom_idx, aa_idx, hyb_idx, cont_feat, atom_table, aa_table, hyb_table):
    fused_tab = jnp.concatenate(
        [
            jnp.broadcast_to(atom_table[:, None, None, :],
                             (N_ATOMS, N_AA, N_HYB, EMB_ATOM)),
            jnp.broadcast_to(aa_table[None, :, None, :],
                             (N_ATOMS, N_AA, N_HYB, EMB_AA)),
            jnp.broadcast_to(hyb_table[None, None, :, :],
                             (N_ATOMS, N_AA, N_HYB, EMB_HYB)),
        ],
        axis=-1,
    ).reshape(FUSED_ROWS, EMB_DIM)
    fused_tab = jnp.pad(fused_tab, ((0, 0), (0, 128 - EMB_DIM)))
    packed = _gather_sc(
        atom_idx.astype(jnp.int32),
        aa_idx.astype(jnp.int32),
        hyb_idx.astype(jnp.int32),
        fused_tab,
    )
    out_t = _concat_call(packed.reshape(N // 8, 8, 128), cont_feat.T)
    return out_t.T


# speedup vs baseline: 7.9915x; 1.0270x over previous
import functools

import jax
import jax.numpy as jnp
from jax import lax
from jax.experimental import pallas as pl
from jax.experimental.pallas import tpu as pltpu
from jax.experimental.pallas import tpu_sc as plsc

N = 100000
CONT_DIM = 32
N_ATOMS = 44
N_AA = 22
N_HYB = 8
EMB_ATOM = 16
EMB_AA = 8
EMB_HYB = 8
EMB_DIM = EMB_ATOM + EMB_AA + EMB_HYB
OUT_DIM = EMB_DIM + CONT_DIM
FUSED_ROWS = N_ATOMS * N_AA * N_HYB

NUM_WORKERS = 32
CHUNK = 400
NUM_CHUNKS = N // CHUNK
CHUNKS_PER_WORKER = -(-NUM_CHUNKS // NUM_WORKERS)
LANES = 16
PACK = 128 // EMB_DIM
PACKED_ROWS = N // PACK
CHUNK_PACKED = CHUNK // PACK

_mesh = plsc.VectorSubcoreMesh(core_axis_name="c", subcore_axis_name="s")


@functools.partial(
    pl.kernel,
    out_type=jax.ShapeDtypeStruct((N, 128), jnp.float32),
    mesh=_mesh,
    scratch_types=[
        pltpu.VMEM((CHUNK,), jnp.int32),
        pltpu.VMEM((CHUNK,), jnp.int32),
        pltpu.VMEM((CHUNK,), jnp.int32),
        pltpu.VMEM((CHUNK, 128), jnp.float32),
        pltpu.VMEM((CHUNK, 128), jnp.float32),
        pltpu.SemaphoreType.DMA,
        pltpu.SemaphoreType.DMA,
        pltpu.SemaphoreType.DMA,
        pltpu.SemaphoreType.DMA,
    ],
    compiler_params=pltpu.CompilerParams(use_tc_tiling_on_sc=False),
)
def _gather_sc(atom_idx, aa_idx, hyb_idx, fused_tab, out,
               ia_v, ib_v, ih_v, o_v0, o_v1, sem_i, sem_g, sem_o0, sem_o1):
    wid = lax.axis_index("s") * 2 + lax.axis_index("c")
    bufs = (o_v0, o_v1)
    osems = (sem_o0, sem_o1)
    pending = [None, None]

    for t in range(CHUNKS_PER_WORKER):
        b = t % 2
        if pending[b] is not None:
            pending[b].wait()
        chunk = jnp.minimum(wid + NUM_WORKERS * t, NUM_CHUNKS - 1)
        base = chunk * CHUNK
        rows = pl.ds(base, CHUNK)
        ca = pltpu.async_copy(atom_idx.at[rows], ia_v, sem_i)
        cb = pltpu.async_copy(aa_idx.at[rows], ib_v, sem_i)
        ch = pltpu.async_copy(hyb_idx.at[rows], ih_v, sem_i)
        ca.wait()
        cb.wait()
        ch.wait()

        def fuse(i, _):
            s = pl.ds(i * LANES, LANES)
            ia_v[s] = (ia_v[s] * (N_AA * N_HYB) + ib_v[s] * N_HYB
                       + ih_v[s])
            return 0

        lax.fori_loop(0, CHUNK // LANES, fuse, 0)

        pltpu.async_copy(fused_tab.at[ia_v], bufs[b], sem_g).wait()
        pending[b] = pltpu.async_copy(bufs[b], out.at[rows], osems[b])
    for p in pending:
        p.wait()


TC_BLOCK = 8192


def _concat_tc(emb_ref, cont_ref, out_ref):
    e = emb_ref[...].reshape(TC_BLOCK, 128)[:, 0:EMB_DIM]
    out_ref[0:EMB_DIM, :] = jnp.transpose(e)
    out_ref[EMB_DIM:OUT_DIM, :] = cont_ref[...]


_concat_call = pl.pallas_call(
    _concat_tc,
    grid=(-(-N // TC_BLOCK),),
    in_specs=[
        pl.BlockSpec((TC_BLOCK // 8, 8, 128), lambda i: (i, 0, 0)),
        pl.BlockSpec((CONT_DIM, TC_BLOCK), lambda i: (0, i)),
    ],
    out_specs=pl.BlockSpec((OUT_DIM, TC_BLOCK), lambda i: (0, i)),
    out_shape=jax.ShapeDtypeStruct((OUT_DIM, N), jnp.float32),
)


def kernel(atom_idx, aa_idx, hyb_idx, cont_feat, atom_table, aa_table, hyb_table):
    fused_tab = jnp.concatenate(
        [
            jnp.broadcast_to(atom_table[:, None, None, :],
                             (N_ATOMS, N_AA, N_HYB, EMB_ATOM)),
            jnp.broadcast_to(aa_table[None, :, None, :],
                             (N_ATOMS, N_AA, N_HYB, EMB_AA)),
            jnp.broadcast_to(hyb_table[None, None, :, :],
                             (N_ATOMS, N_AA, N_HYB, EMB_HYB)),
        ],
        axis=-1,
    ).reshape(FUSED_ROWS, EMB_DIM)
    fused_tab = jnp.pad(fused_tab, ((0, 0), (0, 128 - EMB_DIM)))
    packed = _gather_sc(
        atom_idx.astype(jnp.int32),
        aa_idx.astype(jnp.int32),
        hyb_idx.astype(jnp.int32),
        fused_tab,
    )
    out_t = _concat_call(packed.reshape(N // 8, 8, 128), cont_feat.T)
    return out_t.T
